# baseline (speedup 1.0000x reference)
import functools

import jax
import jax.numpy as jnp
from jax import lax
from jax.experimental import pallas as pl
from jax.experimental.pallas import tpu as pltpu
from jax.experimental.pallas import tpu_sc as plsc

N_ATOMS = 110000
PER_DEG = 10000
D_FEAT = 128
D_OUT = 128
MAX_DEG = 10

NW = 32
CHUNK = 40
NCHUNK = PER_DEG // CHUNK
LANES = 16
NCOL = D_FEAT // LANES


NK = -(-NCHUNK // NW)


def _sc_body(feat_hbm, adj_cat, rel_hbm, *rest):
    idx0, idx1, rows0, rows1, acc_v, si0, si1, sg0, sg1 = rest
    idx = (idx0, idx1)
    rows = (rows0, rows1)
    si = (si0, si1)
    sg = (sg0, sg1)

    cid = lax.axis_index("c")
    sid = lax.axis_index("s")
    wid = sid * 2 + cid

    for d in range(1, MAX_DEG + 1):
        abase = PER_DEG * d * (d - 1) // 2
        nrows = CHUNK * d
        out_base = (d - 1) * PER_DEG

        def fire_idx(k, b, abase=abase, nrows=nrows):
            c = wid + k * NW

            @pl.when(c < NCHUNK)
            def _():
                pltpu.async_copy(adj_cat.at[pl.ds(abase + c * nrows, nrows)],
                                 idx[b].at[pl.ds(0, nrows)], si[b])

        def fire_gather(k, b, abase=abase, nrows=nrows):
            c = wid + k * NW

            @pl.when(c < NCHUNK)
            def _():
                pltpu.make_async_copy(
                    adj_cat.at[pl.ds(abase + c * nrows, nrows)],
                    idx[b].at[pl.ds(0, nrows)],
                    si[b]).wait()
                pltpu.async_copy(feat_hbm.at[idx[b].at[pl.ds(0, nrows)]],
                                 rows[b].at[pl.ds(0, nrows)], sg[b])

        def wait_gather(k, b, nrows=nrows):
            c = wid + k * NW

            @pl.when(c < NCHUNK)
            def _():
                pltpu.make_async_copy(
                    feat_hbm.at[idx[b].at[pl.ds(0, nrows)]],
                    rows[b].at[pl.ds(0, nrows)], sg[b]).wait()

        def sum_store(k, b, d=d, out_base=out_base):
            c = wid + k * NW

            @pl.when(c < NCHUNK)
            def _():
                row0 = c * CHUNK
                if d == 1:
                    pltpu.sync_copy(rows[b].at[pl.ds(0, CHUNK)],
                                    rel_hbm.at[pl.ds(row0, CHUNK)])
                else:
                    def row_body(r, _):
                        base = r * d
                        for cb in range(NCOL):
                            s = cb * LANES
                            v = rows[b][base, pl.ds(s, LANES)]
                            for j in range(1, d):
                                v = v + rows[b][base + j, pl.ds(s, LANES)]
                            acc_v[r, pl.ds(s, LANES)] = v
                        return 0
                    lax.fori_loop(0, CHUNK, row_body, 0)
                    pltpu.sync_copy(acc_v,
                                    rel_hbm.at[pl.ds(out_base + row0, CHUNK)])

        fire_idx(0, 0)
        fire_idx(1, 1)
        fire_gather(0, 0)

        def pipe_body(t, _):
            k0 = 2 * t
            k1 = k0 + 1
            wait_gather(k0, 0)
            fire_idx(k0 + 2, 0)
            fire_gather(k1, 1)
            sum_store(k0, 0)
            fire_gather(k0 + 2, 0)
            wait_gather(k1, 1)
            fire_idx(k1 + 2, 1)
            sum_store(k1, 1)
            return 0

        lax.fori_loop(0, NK // 2, pipe_body, 0)


@jax.jit
def _sc_gather_sum(feat, adj_cat):
    mesh = plsc.VectorSubcoreMesh(core_axis_name="c", subcore_axis_name="s")
    nmax = CHUNK * MAX_DEG
    return pl.kernel(
        _sc_body,
        out_type=jax.ShapeDtypeStruct((MAX_DEG * PER_DEG, D_FEAT),
                                      jnp.float32),
        mesh=mesh,
        scratch_types=[
            pltpu.VMEM((nmax,), jnp.int32),
            pltpu.VMEM((nmax,), jnp.int32),
            pltpu.VMEM((nmax, D_FEAT), jnp.float32),
            pltpu.VMEM((nmax, D_FEAT), jnp.float32),
            pltpu.VMEM((CHUNK, D_FEAT), jnp.float32),
            pltpu.SemaphoreType.DMA,
            pltpu.SemaphoreType.DMA,
            pltpu.SemaphoreType.DMA,
            pltpu.SemaphoreType.DMA,
        ],
    )(feat, adj_cat)


BM = 1000


def _mm_self_body(self_ref, ws_ref, b_ref, o_ref):
    d = pl.program_id(0)
    del d
    o_ref[...] = jnp.dot(self_ref[...], ws_ref[0],
                         preferred_element_type=jnp.float32) + b_ref[0]


def _mm_rel_body(base_ref, rel_ref, wr_ref, o_ref):
    o_ref[...] = base_ref[...] + jnp.dot(rel_ref[...], wr_ref[0],
                                         preferred_element_type=jnp.float32)


@jax.jit
def _tc_self(feat, w_self, b_comb):
    nb = PER_DEG // BM
    return pl.pallas_call(
        _mm_self_body,
        grid=(MAX_DEG + 1, nb),
        in_specs=[
            pl.BlockSpec((BM, D_FEAT), lambda d, i: (d * nb + i, 0)),
            pl.BlockSpec((1, D_FEAT, D_OUT), lambda d, i: (d, 0, 0)),
            pl.BlockSpec((1, 1, D_OUT), lambda d, i: (d, 0, 0)),
        ],
        out_specs=pl.BlockSpec((BM, D_OUT), lambda d, i: (d * nb + i, 0)),
        out_shape=jax.ShapeDtypeStruct((N_ATOMS, D_OUT), jnp.float32),
    )(feat, w_self, b_comb)


@jax.jit
def _tc_rel_add(base, rel, w_rel):
    nb = PER_DEG // BM
    return pl.pallas_call(
        _mm_rel_body,
        grid=(MAX_DEG, nb),
        in_specs=[
            pl.BlockSpec((BM, D_OUT), lambda g, i: ((g + 1) * nb + i, 0)),
            pl.BlockSpec((BM, D_FEAT), lambda g, i: (g * nb + i, 0)),
            pl.BlockSpec((1, D_FEAT, D_OUT), lambda g, i: (g, 0, 0)),
        ],
        out_specs=pl.BlockSpec((BM, D_OUT), lambda g, i: ((g + 1) * nb + i, 0)),
        out_shape=jax.ShapeDtypeStruct((N_ATOMS, D_OUT), jnp.float32),
        input_output_aliases={0: 0},
    )(base, rel, w_rel)


def kernel(atom_features, deg_slice, adj_1, adj_2, adj_3, adj_4, adj_5,
           adj_6, adj_7, adj_8, adj_9, adj_10, W, b):
    del deg_slice
    adjs = [adj_1, adj_2, adj_3, adj_4, adj_5, adj_6, adj_7, adj_8,
            adj_9, adj_10]
    adj_cat = jnp.concatenate([a.reshape(-1) for a in adjs])
    rel = _sc_gather_sum(atom_features, adj_cat)
    w_rel = W[0:20:2]
    w_self = jnp.concatenate([W[20:21], W[1:20:2]], axis=0)
    b_comb = jnp.concatenate([b[20:21], b[0:20:2] + b[1:20:2]], axis=0)
    base = _tc_self(atom_features, w_self,
                    b_comb.reshape(MAX_DEG + 1, 1, D_OUT))
    return _tc_rel_add(base, rel, w_rel)


# speedup vs baseline: 1.0569x; 1.0569x over previous
import functools

import jax
import jax.numpy as jnp
from jax import lax
from jax.experimental import pallas as pl
from jax.experimental.pallas import tpu as pltpu
from jax.experimental.pallas import tpu_sc as plsc

N_ATOMS = 110000
PER_DEG = 10000
D_FEAT = 128
D_OUT = 128
MAX_DEG = 10

NW = 32
CHUNK = 40
NCHUNK = PER_DEG // CHUNK
LANES = 16
NCOL = D_FEAT // LANES


NK = -(-NCHUNK // NW)


def _sc_body(feat_hbm, *rest):
    adj_refs = rest[:MAX_DEG]
    rel_hbm = rest[MAX_DEG]
    idx0, idx1, rows0, rows1, acc_v, si0, si1, sg0, sg1 = rest[MAX_DEG + 1:]
    idx = (idx0, idx1)
    rows = (rows0, rows1)
    si = (si0, si1)
    sg = (sg0, sg1)

    cid = lax.axis_index("c")
    sid = lax.axis_index("s")
    wid = sid * 2 + cid

    for d in range(1, MAX_DEG + 1):
        adj = adj_refs[d - 1]
        nrows = CHUNK * d
        out_base = (d - 1) * PER_DEG

        def fire_idx(k, b, adj=adj, nrows=nrows):
            c = wid + k * NW

            @pl.when(c < NCHUNK)
            def _():
                pltpu.async_copy(adj.at[pl.ds(c * nrows, nrows)],
                                 idx[b].at[pl.ds(0, nrows)], si[b])

        def fire_gather(k, b, adj=adj, nrows=nrows):
            c = wid + k * NW

            @pl.when(c < NCHUNK)
            def _():
                pltpu.make_async_copy(adj.at[pl.ds(c * nrows, nrows)],
                                      idx[b].at[pl.ds(0, nrows)],
                                      si[b]).wait()
                pltpu.async_copy(feat_hbm.at[idx[b].at[pl.ds(0, nrows)]],
                                 rows[b].at[pl.ds(0, nrows)], sg[b])

        def wait_gather(k, b, nrows=nrows):
            c = wid + k * NW

            @pl.when(c < NCHUNK)
            def _():
                pltpu.make_async_copy(
                    feat_hbm.at[idx[b].at[pl.ds(0, nrows)]],
                    rows[b].at[pl.ds(0, nrows)], sg[b]).wait()

        def sum_store(k, b, d=d, out_base=out_base):
            c = wid + k * NW

            @pl.when(c < NCHUNK)
            def _():
                row0 = c * CHUNK
                if d == 1:
                    pltpu.sync_copy(rows[b].at[pl.ds(0, CHUNK)],
                                    rel_hbm.at[pl.ds(row0, CHUNK)])
                else:
                    def row_body(r, _):
                        base = r * d
                        for cb in range(NCOL):
                            s = cb * LANES
                            v = rows[b][base, pl.ds(s, LANES)]
                            for j in range(1, d):
                                v = v + rows[b][base + j, pl.ds(s, LANES)]
                            acc_v[r, pl.ds(s, LANES)] = v
                        return 0
                    lax.fori_loop(0, CHUNK, row_body, 0)
                    pltpu.sync_copy(acc_v,
                                    rel_hbm.at[pl.ds(out_base + row0, CHUNK)])

        fire_idx(0, 0)
        fire_idx(1, 1)
        fire_gather(0, 0)

        def pipe_body(t, _):
            k0 = 2 * t
            k1 = k0 + 1
            wait_gather(k0, 0)
            fire_idx(k0 + 2, 0)
            fire_gather(k1, 1)
            sum_store(k0, 0)
            fire_gather(k0 + 2, 0)
            wait_gather(k1, 1)
            fire_idx(k1 + 2, 1)
            sum_store(k1, 1)
            return 0

        lax.fori_loop(0, NK // 2, pipe_body, 0)


@jax.jit
def _sc_gather_sum(feat, *adj_flat):
    mesh = plsc.VectorSubcoreMesh(core_axis_name="c", subcore_axis_name="s")
    nmax = CHUNK * MAX_DEG
    return pl.kernel(
        _sc_body,
        out_type=jax.ShapeDtypeStruct((MAX_DEG * PER_DEG, D_FEAT),
                                      jnp.float32),
        mesh=mesh,
        scratch_types=[
            pltpu.VMEM((nmax,), jnp.int32),
            pltpu.VMEM((nmax,), jnp.int32),
            pltpu.VMEM((nmax, D_FEAT), jnp.float32),
            pltpu.VMEM((nmax, D_FEAT), jnp.float32),
            pltpu.VMEM((CHUNK, D_FEAT), jnp.float32),
            pltpu.SemaphoreType.DMA,
            pltpu.SemaphoreType.DMA,
            pltpu.SemaphoreType.DMA,
            pltpu.SemaphoreType.DMA,
        ],
    )(feat, *adj_flat)


BM = 1000


def _mm_self_body(self_ref, ws_ref, b_ref, o_ref):
    d = pl.program_id(0)
    del d
    o_ref[...] = jnp.dot(self_ref[...], ws_ref[0],
                         preferred_element_type=jnp.float32) + b_ref[0]


def _mm_rel_body(base_ref, rel_ref, wr_ref, o_ref):
    o_ref[...] = base_ref[...] + jnp.dot(
        rel_ref[...].astype(jnp.bfloat16), wr_ref[0],
        preferred_element_type=jnp.float32)


@jax.jit
def _tc_self(feat, w_self, b_comb):
    nb = PER_DEG // BM
    return pl.pallas_call(
        _mm_self_body,
        grid=(MAX_DEG + 1, nb),
        in_specs=[
            pl.BlockSpec((BM, D_FEAT), lambda d, i: (d * nb + i, 0)),
            pl.BlockSpec((1, D_FEAT, D_OUT), lambda d, i: (d, 0, 0)),
            pl.BlockSpec((1, 1, D_OUT), lambda d, i: (d, 0, 0)),
        ],
        out_specs=pl.BlockSpec((BM, D_OUT), lambda d, i: (d * nb + i, 0)),
        out_shape=jax.ShapeDtypeStruct((N_ATOMS, D_OUT), jnp.float32),
    )(feat, w_self, b_comb)


@jax.jit
def _tc_rel_add(base, rel, w_rel):
    nb = PER_DEG // BM
    return pl.pallas_call(
        _mm_rel_body,
        grid=(MAX_DEG, nb),
        in_specs=[
            pl.BlockSpec((BM, D_OUT), lambda g, i: ((g + 1) * nb + i, 0)),
            pl.BlockSpec((BM, D_FEAT), lambda g, i: (g * nb + i, 0)),
            pl.BlockSpec((1, D_FEAT, D_OUT), lambda g, i: (g, 0, 0)),
        ],
        out_specs=pl.BlockSpec((BM, D_OUT), lambda g, i: ((g + 1) * nb + i, 0)),
        out_shape=jax.ShapeDtypeStruct((N_ATOMS, D_OUT), jnp.float32),
        input_output_aliases={0: 0},
    )(base, rel, w_rel)


def kernel(atom_features, deg_slice, adj_1, adj_2, adj_3, adj_4, adj_5,
           adj_6, adj_7, adj_8, adj_9, adj_10, W, b):
    del deg_slice
    adjs = [adj_1, adj_2, adj_3, adj_4, adj_5, adj_6, adj_7, adj_8,
            adj_9, adj_10]
    rel = _sc_gather_sum(atom_features,
                         *[a.reshape(-1) for a in adjs])
    w_rel = W[0:20:2]
    w_self = jnp.concatenate([W[20:21], W[1:20:2]], axis=0)
    b_comb = jnp.concatenate([b[20:21], b[0:20:2] + b[1:20:2]], axis=0)
    base = _tc_self(atom_features, w_self,
                    b_comb.reshape(MAX_DEG + 1, 1, D_OUT))
    return _tc_rel_add(base, rel, w_rel.astype(jnp.bfloat16))


# speedup vs baseline: 1.3179x; 1.2469x over previous
import functools

import jax
import jax.numpy as jnp
from jax import lax
from jax.experimental import pallas as pl
from jax.experimental.pallas import tpu as pltpu
from jax.experimental.pallas import tpu_sc as plsc

N_ATOMS = 110000
PER_DEG = 10000
D_FEAT = 128
D_OUT = 128
MAX_DEG = 10

NW = 32
CHUNK = 40
NCHUNK = PER_DEG // CHUNK
LANES = 16
NCOL = D_FEAT // LANES


NK = -(-NCHUNK // NW)


def _sc_body(lo, hi, feat_hbm, *rest):
    nd = hi - lo + 1
    adj_refs = rest[:nd]
    rel_hbm = rest[nd]
    idx0, idx1, rows0, rows1, acc_v, si0, si1, sg0, sg1 = rest[nd + 1:]
    idx = (idx0, idx1)
    rows = (rows0, rows1)
    si = (si0, si1)
    sg = (sg0, sg1)

    cid = lax.axis_index("c")
    sid = lax.axis_index("s")
    wid = sid * 2 + cid

    for d in range(lo, hi + 1):
        adj = adj_refs[d - lo]
        nrows = CHUNK * d
        out_base = (d - lo) * PER_DEG

        def fire_idx(k, b, adj=adj, nrows=nrows):
            c = wid + k * NW

            @pl.when(c < NCHUNK)
            def _():
                pltpu.async_copy(adj.at[pl.ds(c * nrows, nrows)],
                                 idx[b].at[pl.ds(0, nrows)], si[b])

        def fire_gather(k, b, adj=adj, nrows=nrows):
            c = wid + k * NW

            @pl.when(c < NCHUNK)
            def _():
                pltpu.make_async_copy(adj.at[pl.ds(c * nrows, nrows)],
                                      idx[b].at[pl.ds(0, nrows)],
                                      si[b]).wait()
                pltpu.async_copy(feat_hbm.at[idx[b].at[pl.ds(0, nrows)]],
                                 rows[b].at[pl.ds(0, nrows)], sg[b])

        def wait_gather(k, b, nrows=nrows):
            c = wid + k * NW

            @pl.when(c < NCHUNK)
            def _():
                pltpu.make_async_copy(
                    feat_hbm.at[idx[b].at[pl.ds(0, nrows)]],
                    rows[b].at[pl.ds(0, nrows)], sg[b]).wait()

        def sum_store(k, b, d=d, out_base=out_base):
            c = wid + k * NW

            @pl.when(c < NCHUNK)
            def _():
                row0 = c * CHUNK
                if d == 1:
                    pltpu.sync_copy(rows[b].at[pl.ds(0, CHUNK)],
                                    rel_hbm.at[pl.ds(row0, CHUNK)])
                else:
                    def row_body(r, _):
                        base = r * d
                        for cb in range(NCOL):
                            s = cb * LANES
                            v = rows[b][base, pl.ds(s, LANES)]
                            for j in range(1, d):
                                v = v + rows[b][base + j, pl.ds(s, LANES)]
                            acc_v[r, pl.ds(s, LANES)] = v
                        return 0
                    lax.fori_loop(0, CHUNK, row_body, 0)
                    pltpu.sync_copy(acc_v,
                                    rel_hbm.at[pl.ds(out_base + row0, CHUNK)])

        fire_idx(0, 0)
        fire_idx(1, 1)
        fire_gather(0, 0)

        def pipe_body(t, _):
            k0 = 2 * t
            k1 = k0 + 1
            wait_gather(k0, 0)
            fire_idx(k0 + 2, 0)
            fire_gather(k1, 1)
            sum_store(k0, 0)
            fire_gather(k0 + 2, 0)
            wait_gather(k1, 1)
            fire_idx(k1 + 2, 1)
            sum_store(k1, 1)
            return 0

        lax.fori_loop(0, NK // 2, pipe_body, 0)


@functools.partial(jax.jit, static_argnums=(0, 1))
def _sc_gather_sum(lo, hi, feat, *adj_flat):
    mesh = plsc.VectorSubcoreMesh(core_axis_name="c", subcore_axis_name="s")
    nmax = CHUNK * hi
    nd = hi - lo + 1
    return pl.kernel(
        functools.partial(_sc_body, lo, hi),
        out_type=jax.ShapeDtypeStruct((nd * PER_DEG, D_FEAT),
                                      jnp.float32),
        mesh=mesh,
        scratch_types=[
            pltpu.VMEM((nmax,), jnp.int32),
            pltpu.VMEM((nmax,), jnp.int32),
            pltpu.VMEM((nmax, D_FEAT), jnp.float32),
            pltpu.VMEM((nmax, D_FEAT), jnp.float32),
            pltpu.VMEM((CHUNK, D_FEAT), jnp.float32),
            pltpu.SemaphoreType.DMA,
            pltpu.SemaphoreType.DMA,
            pltpu.SemaphoreType.DMA,
            pltpu.SemaphoreType.DMA,
        ],
    )(feat, *adj_flat)


BM = 1000


def _mm_self_body(self_ref, ws_ref, b_ref, o_ref):
    d = pl.program_id(0)
    del d
    o_ref[...] = jnp.dot(self_ref[...], ws_ref[0],
                         preferred_element_type=jnp.float32) + b_ref[0]


def _mm_rel_body(base_ref, rel_ref, wr_ref, o_ref):
    o_ref[...] = base_ref[...] + jnp.dot(
        rel_ref[...].astype(jnp.bfloat16), wr_ref[0],
        preferred_element_type=jnp.float32)


@jax.jit
def _tc_self(feat, w_self, b_comb):
    nb = PER_DEG // BM
    return pl.pallas_call(
        _mm_self_body,
        grid=(MAX_DEG + 1, nb),
        in_specs=[
            pl.BlockSpec((BM, D_FEAT), lambda d, i: (d * nb + i, 0)),
            pl.BlockSpec((1, D_FEAT, D_OUT), lambda d, i: (d, 0, 0)),
            pl.BlockSpec((1, 1, D_OUT), lambda d, i: (d, 0, 0)),
        ],
        out_specs=pl.BlockSpec((BM, D_OUT), lambda d, i: (d * nb + i, 0)),
        out_shape=jax.ShapeDtypeStruct((N_ATOMS, D_OUT), jnp.float32),
    )(feat, w_self, b_comb)


@functools.partial(jax.jit, static_argnums=(3, 4))
def _tc_rel_add(base, rel, w_rel, lo, hi):
    nb = PER_DEG // BM
    nd = hi - lo + 1
    return pl.pallas_call(
        _mm_rel_body,
        grid=(nd, nb),
        in_specs=[
            pl.BlockSpec((BM, D_OUT), lambda g, i: ((lo + g) * nb + i, 0)),
            pl.BlockSpec((BM, D_FEAT), lambda g, i: (g * nb + i, 0)),
            pl.BlockSpec((1, D_FEAT, D_OUT), lambda g, i: (g, 0, 0)),
        ],
        out_specs=pl.BlockSpec((BM, D_OUT),
                               lambda g, i: ((lo + g) * nb + i, 0)),
        out_shape=jax.ShapeDtypeStruct((N_ATOMS, D_OUT), jnp.float32),
        input_output_aliases={0: 0},
    )(base, rel, w_rel)


GROUPS = [(9, 10), (6, 8), (2, 5), (1, 1)]


def kernel(atom_features, deg_slice, adj_1, adj_2, adj_3, adj_4, adj_5,
           adj_6, adj_7, adj_8, adj_9, adj_10, W, b):
    del deg_slice
    adjs = [adj_1, adj_2, adj_3, adj_4, adj_5, adj_6, adj_7, adj_8,
            adj_9, adj_10]
    w_rel = W[0:20:2].astype(jnp.bfloat16)
    w_self = jnp.concatenate([W[20:21], W[1:20:2]], axis=0)
    b_comb = jnp.concatenate([b[20:21], b[0:20:2] + b[1:20:2]], axis=0)
    out = _tc_self(atom_features, w_self,
                   b_comb.reshape(MAX_DEG + 1, 1, D_OUT))
    rels = [(lo, hi,
             _sc_gather_sum(lo, hi, atom_features,
                            *[adjs[d - 1].reshape(-1)
                              for d in range(lo, hi + 1)]))
            for lo, hi in GROUPS]
    for lo, hi, rel_g in rels:
        out = _tc_rel_add(out, rel_g, w_rel[lo - 1:hi], lo, hi)
    return out


# speedup vs baseline: 1.4075x; 1.0680x over previous
import functools

import jax
import jax.numpy as jnp
from jax import lax
from jax.experimental import pallas as pl
from jax.experimental.pallas import tpu as pltpu
from jax.experimental.pallas import tpu_sc as plsc

N_ATOMS = 110000
PER_DEG = 10000
D_FEAT = 128
D_OUT = 128
MAX_DEG = 10

NW = 32
CHUNK = 40
NCHUNK = PER_DEG // CHUNK
LANES = 16
NCOL = D_FEAT // LANES


NK = -(-NCHUNK // NW)


def _sc_body(lo, hi, feat_hbm, *rest):
    nd = hi - lo + 1
    adj_refs = rest[:nd]
    rel_hbm = rest[nd]
    (idx0, idx1, rows0, rows1, acc0, acc1,
     si0, si1, sg0, sg1, so0, so1) = rest[nd + 1:]
    idx = (idx0, idx1)
    rows = (rows0, rows1)
    acc = (acc0, acc1)
    si = (si0, si1)
    sg = (sg0, sg1)
    so = (so0, so1)

    cid = lax.axis_index("c")
    sid = lax.axis_index("s")
    wid = sid * 2 + cid

    def drain_out(b):
        pltpu.make_async_copy(acc[b], rel_hbm.at[pl.ds(0, CHUNK)],
                              so[b]).wait()

    any_acc = False
    for d in range(lo, hi + 1):
        first_deg = d == lo
        any_acc = any_acc or d > 1
        adj = adj_refs[d - lo]
        nrows = CHUNK * d
        out_base = (d - lo) * PER_DEG

        def fire_idx(k, b, adj=adj, nrows=nrows):
            c = wid + k * NW

            @pl.when(c < NCHUNK)
            def _():
                pltpu.async_copy(adj.at[pl.ds(c * nrows, nrows)],
                                 idx[b].at[pl.ds(0, nrows)], si[b])

        def fire_gather(k, b, adj=adj, nrows=nrows):
            c = wid + k * NW

            @pl.when(c < NCHUNK)
            def _():
                pltpu.make_async_copy(adj.at[pl.ds(c * nrows, nrows)],
                                      idx[b].at[pl.ds(0, nrows)],
                                      si[b]).wait()
                pltpu.async_copy(feat_hbm.at[idx[b].at[pl.ds(0, nrows)]],
                                 rows[b].at[pl.ds(0, nrows)], sg[b])

        def wait_gather(k, b, nrows=nrows):
            c = wid + k * NW

            @pl.when(c < NCHUNK)
            def _():
                pltpu.make_async_copy(
                    feat_hbm.at[idx[b].at[pl.ds(0, nrows)]],
                    rows[b].at[pl.ds(0, nrows)], sg[b]).wait()

        def sum_store(k, b, d=d, out_base=out_base, first_deg=first_deg):
            c = wid + k * NW

            @pl.when(c < NCHUNK)
            def _():
                row0 = c * CHUNK
                if d == 1:
                    pltpu.sync_copy(rows[b].at[pl.ds(0, CHUNK)],
                                    rel_hbm.at[pl.ds(row0, CHUNK)])
                else:
                    if first_deg:
                        @pl.when(k >= 2)
                        def _():
                            drain_out(b)
                    else:
                        drain_out(b)

                    def row_body(m, _):
                        for half in range(2):
                            r = 2 * m + half
                            base = r * d
                            for cb in range(NCOL):
                                s = cb * LANES
                                v = rows[b][base, pl.ds(s, LANES)]
                                for j in range(1, d):
                                    v = v + rows[b][base + j,
                                                    pl.ds(s, LANES)]
                                acc[b][r, pl.ds(s, LANES)] = v
                        return 0
                    lax.fori_loop(0, CHUNK // 2, row_body, 0)
                    pltpu.async_copy(
                        acc[b],
                        rel_hbm.at[pl.ds(out_base + row0, CHUNK)], so[b])

        fire_idx(0, 0)
        fire_idx(1, 1)
        fire_gather(0, 0)

        def pipe_body(t, _):
            k0 = 2 * t
            k1 = k0 + 1
            wait_gather(k0, 0)
            fire_idx(k0 + 2, 0)
            fire_gather(k1, 1)
            sum_store(k0, 0)
            fire_gather(k0 + 2, 0)
            wait_gather(k1, 1)
            fire_idx(k1 + 2, 1)
            sum_store(k1, 1)
            return 0

        lax.fori_loop(0, NK // 2, pipe_body, 0)

    if any_acc:
        drain_out(0)
        drain_out(1)


@functools.partial(jax.jit, static_argnums=(0, 1))
def _sc_gather_sum(lo, hi, feat, *adj_flat):
    mesh = plsc.VectorSubcoreMesh(core_axis_name="c", subcore_axis_name="s")
    nmax = CHUNK * hi
    nd = hi - lo + 1
    return pl.kernel(
        functools.partial(_sc_body, lo, hi),
        out_type=jax.ShapeDtypeStruct((nd * PER_DEG, D_FEAT),
                                      jnp.float32),
        mesh=mesh,
        scratch_types=[
            pltpu.VMEM((nmax,), jnp.int32),
            pltpu.VMEM((nmax,), jnp.int32),
            pltpu.VMEM((nmax, D_FEAT), jnp.float32),
            pltpu.VMEM((nmax, D_FEAT), jnp.float32),
            pltpu.VMEM((CHUNK, D_FEAT), jnp.float32),
            pltpu.VMEM((CHUNK, D_FEAT), jnp.float32),
            pltpu.SemaphoreType.DMA,
            pltpu.SemaphoreType.DMA,
            pltpu.SemaphoreType.DMA,
            pltpu.SemaphoreType.DMA,
            pltpu.SemaphoreType.DMA,
            pltpu.SemaphoreType.DMA,
        ],
    )(feat, *adj_flat)


BM = 1000


def _mm_self_body(self_ref, ws_ref, b_ref, o_ref):
    d = pl.program_id(0)
    del d
    o_ref[...] = jnp.dot(self_ref[...], ws_ref[0],
                         preferred_element_type=jnp.float32) + b_ref[0]


def _mm_rel_body(base_ref, rel_ref, wr_ref, o_ref):
    o_ref[...] = base_ref[...] + jnp.dot(
        rel_ref[...].astype(jnp.bfloat16), wr_ref[0],
        preferred_element_type=jnp.float32)


@jax.jit
def _tc_self(feat, w_self, b_comb):
    nb = PER_DEG // BM
    return pl.pallas_call(
        _mm_self_body,
        grid=(MAX_DEG + 1, nb),
        in_specs=[
            pl.BlockSpec((BM, D_FEAT), lambda d, i: (d * nb + i, 0)),
            pl.BlockSpec((1, D_FEAT, D_OUT), lambda d, i: (d, 0, 0)),
            pl.BlockSpec((1, 1, D_OUT), lambda d, i: (d, 0, 0)),
        ],
        out_specs=pl.BlockSpec((BM, D_OUT), lambda d, i: (d * nb + i, 0)),
        out_shape=jax.ShapeDtypeStruct((N_ATOMS, D_OUT), jnp.float32),
    )(feat, w_self, b_comb)


@functools.partial(jax.jit, static_argnums=(3, 4))
def _tc_rel_add(base, rel, w_rel, lo, hi):
    nb = PER_DEG // BM
    nd = hi - lo + 1
    return pl.pallas_call(
        _mm_rel_body,
        grid=(nd, nb),
        in_specs=[
            pl.BlockSpec((BM, D_OUT), lambda g, i: ((lo + g) * nb + i, 0)),
            pl.BlockSpec((BM, D_FEAT), lambda g, i: (g * nb + i, 0)),
            pl.BlockSpec((1, D_FEAT, D_OUT), lambda g, i: (g, 0, 0)),
        ],
        out_specs=pl.BlockSpec((BM, D_OUT),
                               lambda g, i: ((lo + g) * nb + i, 0)),
        out_shape=jax.ShapeDtypeStruct((N_ATOMS, D_OUT), jnp.float32),
        input_output_aliases={0: 0},
    )(base, rel, w_rel)


GROUPS = [(6, 8), (2, 5), (1, 1), (9, 9), (10, 10)]


def kernel(atom_features, deg_slice, adj_1, adj_2, adj_3, adj_4, adj_5,
           adj_6, adj_7, adj_8, adj_9, adj_10, W, b):
    del deg_slice
    adjs = [adj_1, adj_2, adj_3, adj_4, adj_5, adj_6, adj_7, adj_8,
            adj_9, adj_10]
    w_rel = W[0:20:2].astype(jnp.bfloat16)
    w_self = jnp.concatenate([W[20:21], W[1:20:2]], axis=0)
    b_comb = jnp.concatenate([b[20:21], b[0:20:2] + b[1:20:2]], axis=0)
    out = _tc_self(atom_features, w_self,
                   b_comb.reshape(MAX_DEG + 1, 1, D_OUT))
    rels = [(lo, hi,
             _sc_gather_sum(lo, hi, atom_features,
                            *[adjs[d - 1].reshape(-1)
                              for d in range(lo, hi + 1)]))
            for lo, hi in GROUPS]
    for lo, hi, rel_g in rels:
        out = _tc_rel_add(out, rel_g, w_rel[lo - 1:hi], lo, hi)
    return out


# speedup vs baseline: 1.4609x; 1.0379x over previous
import functools

import jax
import jax.numpy as jnp
from jax import lax
from jax.experimental import pallas as pl
from jax.experimental.pallas import tpu as pltpu
from jax.experimental.pallas import tpu_sc as plsc

N_ATOMS = 110000
PER_DEG = 10000
D_FEAT = 128
D_OUT = 128
MAX_DEG = 10

NW = 32
CHUNK = 40
NCHUNK = PER_DEG // CHUNK
LANES = 16
NCOL = D_FEAT // LANES


NK = -(-NCHUNK // NW)


def _sc_body(lo, hi, feat_hbm, *rest):
    nd = hi - lo + 1
    adj_refs = rest[:nd]
    rel_hbm = rest[nd]
    (idx0, idx1, rows0, rows1, acc0, acc1,
     si0, si1, sg0, sg1, so0, so1) = rest[nd + 1:]
    idx = (idx0, idx1)
    rows = (rows0, rows1)
    acc = (acc0, acc1)
    si = (si0, si1)
    sg = (sg0, sg1)
    so = (so0, so1)

    cid = lax.axis_index("c")
    sid = lax.axis_index("s")
    wid = sid * 2 + cid

    def drain_out(b):
        pltpu.make_async_copy(acc[b], rel_hbm.at[pl.ds(0, CHUNK)],
                              so[b]).wait()

    any_acc = False
    for d in range(lo, hi + 1):
        first_deg = d == lo
        any_acc = any_acc or d > 1
        adj = adj_refs[d - lo]
        nrows = CHUNK * d
        out_base = (d - lo) * PER_DEG

        def fire_idx(k, b, adj=adj, nrows=nrows):
            c = wid + k * NW

            @pl.when(c < NCHUNK)
            def _():
                pltpu.async_copy(adj.at[pl.ds(c * nrows, nrows)],
                                 idx[b].at[pl.ds(0, nrows)], si[b])

        def fire_gather(k, b, adj=adj, nrows=nrows):
            c = wid + k * NW

            @pl.when(c < NCHUNK)
            def _():
                pltpu.make_async_copy(adj.at[pl.ds(c * nrows, nrows)],
                                      idx[b].at[pl.ds(0, nrows)],
                                      si[b]).wait()
                pltpu.async_copy(feat_hbm.at[idx[b].at[pl.ds(0, nrows)]],
                                 rows[b].at[pl.ds(0, nrows)], sg[b])

        def wait_gather(k, b, nrows=nrows):
            c = wid + k * NW

            @pl.when(c < NCHUNK)
            def _():
                pltpu.make_async_copy(
                    feat_hbm.at[idx[b].at[pl.ds(0, nrows)]],
                    rows[b].at[pl.ds(0, nrows)], sg[b]).wait()

        def sum_store(k, b, d=d, out_base=out_base, first_deg=first_deg):
            c = wid + k * NW

            @pl.when(c < NCHUNK)
            def _():
                row0 = c * CHUNK
                if d == 1:
                    pltpu.sync_copy(rows[b].at[pl.ds(0, CHUNK)],
                                    rel_hbm.at[pl.ds(row0, CHUNK)])
                else:
                    if first_deg:
                        @pl.when(k >= 2)
                        def _():
                            drain_out(b)
                    else:
                        drain_out(b)

                    def row_body(m, _):
                        for half in range(2):
                            r = 2 * m + half
                            base = r * d
                            for cb in range(NCOL):
                                s = cb * LANES
                                v = rows[b][base, pl.ds(s, LANES)]
                                for j in range(1, d):
                                    v = v + rows[b][base + j,
                                                    pl.ds(s, LANES)]
                                acc[b][r, pl.ds(s, LANES)] = v
                        return 0
                    lax.fori_loop(0, CHUNK // 2, row_body, 0)
                    pltpu.async_copy(
                        acc[b],
                        rel_hbm.at[pl.ds(out_base + row0, CHUNK)], so[b])

        fire_idx(0, 0)
        fire_idx(1, 1)
        fire_gather(0, 0)

        def pipe_body(t, _):
            k0 = 2 * t
            k1 = k0 + 1
            wait_gather(k0, 0)
            fire_idx(k0 + 2, 0)
            fire_gather(k1, 1)
            sum_store(k0, 0)
            fire_gather(k0 + 2, 0)
            wait_gather(k1, 1)
            fire_idx(k1 + 2, 1)
            sum_store(k1, 1)
            return 0

        lax.fori_loop(0, NK // 2, pipe_body, 0)

    if any_acc:
        drain_out(0)
        drain_out(1)


@functools.partial(jax.jit, static_argnums=(0, 1))
def _sc_gather_sum(lo, hi, feat, *adj_flat):
    mesh = plsc.VectorSubcoreMesh(core_axis_name="c", subcore_axis_name="s")
    nmax = CHUNK * hi
    nd = hi - lo + 1
    return pl.kernel(
        functools.partial(_sc_body, lo, hi),
        out_type=jax.ShapeDtypeStruct((nd * PER_DEG, D_FEAT),
                                      jnp.float32),
        mesh=mesh,
        scratch_types=[
            pltpu.VMEM((nmax,), jnp.int32),
            pltpu.VMEM((nmax,), jnp.int32),
            pltpu.VMEM((nmax, D_FEAT), jnp.float32),
            pltpu.VMEM((nmax, D_FEAT), jnp.float32),
            pltpu.VMEM((CHUNK, D_FEAT), jnp.float32),
            pltpu.VMEM((CHUNK, D_FEAT), jnp.float32),
            pltpu.SemaphoreType.DMA,
            pltpu.SemaphoreType.DMA,
            pltpu.SemaphoreType.DMA,
            pltpu.SemaphoreType.DMA,
            pltpu.SemaphoreType.DMA,
            pltpu.SemaphoreType.DMA,
        ],
    )(feat, *adj_flat)


BM = 1000


def _mm_self_body(self_ref, ws_ref, b_ref, o_ref):
    d = pl.program_id(0)
    del d
    o_ref[...] = jnp.dot(self_ref[...], ws_ref[0],
                         preferred_element_type=jnp.float32) + b_ref[0]


def _mm_rel_body(base_ref, rel_ref, wr_ref, o_ref):
    o_ref[...] = base_ref[...] + jnp.dot(
        rel_ref[...].astype(jnp.bfloat16), wr_ref[0],
        preferred_element_type=jnp.float32)


@jax.jit
def _tc_self(feat, w_self, b_comb):
    nb = PER_DEG // BM
    return pl.pallas_call(
        _mm_self_body,
        grid=(MAX_DEG + 1, nb),
        in_specs=[
            pl.BlockSpec((BM, D_FEAT), lambda d, i: (d * nb + i, 0)),
            pl.BlockSpec((1, D_FEAT, D_OUT), lambda d, i: (d, 0, 0)),
            pl.BlockSpec((1, 1, D_OUT), lambda d, i: (d, 0, 0)),
        ],
        out_specs=pl.BlockSpec((BM, D_OUT), lambda d, i: (d * nb + i, 0)),
        out_shape=jax.ShapeDtypeStruct((N_ATOMS, D_OUT), jnp.float32),
    )(feat, w_self, b_comb)


@functools.partial(jax.jit, static_argnums=(3, 4))
def _tc_rel_add(base, rel, w_rel, lo, hi):
    nb = PER_DEG // BM
    nd = hi - lo + 1
    return pl.pallas_call(
        _mm_rel_body,
        grid=(nd, nb),
        in_specs=[
            pl.BlockSpec((BM, D_OUT), lambda g, i: ((lo + g) * nb + i, 0)),
            pl.BlockSpec((BM, D_FEAT), lambda g, i: (g * nb + i, 0)),
            pl.BlockSpec((1, D_FEAT, D_OUT), lambda g, i: (g, 0, 0)),
        ],
        out_specs=pl.BlockSpec((BM, D_OUT),
                               lambda g, i: ((lo + g) * nb + i, 0)),
        out_shape=jax.ShapeDtypeStruct((N_ATOMS, D_OUT), jnp.float32),
        input_output_aliases={0: 0},
    )(base, rel, w_rel)


GROUPS = [(1, 1), (6, 8), (2, 5), (9, 9), (10, 10)]


def kernel(atom_features, deg_slice, adj_1, adj_2, adj_3, adj_4, adj_5,
           adj_6, adj_7, adj_8, adj_9, adj_10, W, b):
    del deg_slice
    adjs = [adj_1, adj_2, adj_3, adj_4, adj_5, adj_6, adj_7, adj_8,
            adj_9, adj_10]
    w_rel = W[0:20:2].astype(jnp.bfloat16)
    w_self = jnp.concatenate([W[20:21], W[1:20:2]], axis=0)
    b_comb = jnp.concatenate([b[20:21], b[0:20:2] + b[1:20:2]], axis=0)
    out = _tc_self(atom_features, w_self,
                   b_comb.reshape(MAX_DEG + 1, 1, D_OUT))
    rels = [(lo, hi,
             _sc_gather_sum(lo, hi, atom_features,
                            *[adjs[d - 1].reshape(-1)
                              for d in range(lo, hi + 1)]))
            for lo, hi in GROUPS]
    for lo, hi, rel_g in rels:
        out = _tc_rel_add(out, rel_g, w_rel[lo - 1:hi], lo, hi)
    return out
